# baseline (speedup 1.0000x reference)
import functools

import jax
import jax.numpy as jnp
from jax.experimental import pallas as pl

_PLACEHOLDER_TOKEN = 500


def _select_kernel(tok_ref, emb_ref, ph_ref, out_ref):
    mask = tok_ref[...] == _PLACEHOLDER_TOKEN
    out_ref[...] = jnp.where(mask, ph_ref[...], emb_ref[...])


@functools.partial(jax.jit, static_argnames=("block_r",))
def _run(tokenized_text, embedded_text, placeholder_embedding, block_r=1232):
    B, N, D = embedded_text.shape
    R = B * N
    emb2d = embedded_text.reshape(R, D)
    tok2d = tokenized_text.reshape(R, 1)
    grid = (R // block_r,)
    out = pl.pallas_call(
        _select_kernel,
        grid=grid,
        in_specs=[
            pl.BlockSpec((block_r, 1), lambda i: (i, 0)),
            pl.BlockSpec((block_r, D), lambda i: (i, 0)),
            pl.BlockSpec((1, D), lambda i: (0, 0)),
        ],
        out_specs=pl.BlockSpec((block_r, D), lambda i: (i, 0)),
        out_shape=jax.ShapeDtypeStruct((R, D), embedded_text.dtype),
    )(tok2d, emb2d, placeholder_embedding)
    return out.reshape(B, N, D)


def kernel(tokenized_text, embedded_text, placeholder_embedding):
    return _run(tokenized_text, embedded_text, placeholder_embedding)


# speedup vs baseline: 1.5859x; 1.5859x over previous
import functools

import jax
import jax.numpy as jnp
from jax.experimental import pallas as pl

_PLACEHOLDER_TOKEN = 500


def _select_kernel(tok_ref, emb_ref, ph_ref, out_ref):
    mask = tok_ref[...] == _PLACEHOLDER_TOKEN
    out_ref[...] = jnp.where(mask, ph_ref[...], emb_ref[...])


@functools.partial(jax.jit, static_argnames=("block_b",))
def _run(tokenized_text, embedded_text, placeholder_embedding, block_b=8):
    B, N, D = embedded_text.shape
    tok3 = tokenized_text.reshape(B, N, 1)
    ph3 = placeholder_embedding.reshape(1, 1, D)
    grid = (B // block_b,)
    return pl.pallas_call(
        _select_kernel,
        grid=grid,
        in_specs=[
            pl.BlockSpec((block_b, N, 1), lambda i: (i, 0, 0)),
            pl.BlockSpec((block_b, N, D), lambda i: (i, 0, 0)),
            pl.BlockSpec((1, 1, D), lambda i: (0, 0, 0)),
        ],
        out_specs=pl.BlockSpec((block_b, N, D), lambda i: (i, 0, 0)),
        out_shape=jax.ShapeDtypeStruct((B, N, D), embedded_text.dtype),
    )(tok3, embedded_text, ph3)


def kernel(tokenized_text, embedded_text, placeholder_embedding):
    return _run(tokenized_text, embedded_text, placeholder_embedding)


# speedup vs baseline: 1.6220x; 1.0228x over previous
import functools

import jax
import jax.numpy as jnp
from jax.experimental import pallas as pl

_PLACEHOLDER_TOKEN = 500


def _select_kernel(tok_ref, emb_ref, ph_ref, out_ref):
    mask = tok_ref[...] == _PLACEHOLDER_TOKEN
    out_ref[...] = jnp.where(mask, ph_ref[...], emb_ref[...])


@functools.partial(jax.jit, static_argnames=("block_b",))
def _run(tokenized_text, embedded_text, placeholder_embedding, block_b=16):
    B, N, D = embedded_text.shape
    tok3 = tokenized_text.reshape(B, N, 1)
    ph3 = placeholder_embedding.reshape(1, 1, D)
    grid = (B // block_b,)
    return pl.pallas_call(
        _select_kernel,
        grid=grid,
        in_specs=[
            pl.BlockSpec((block_b, N, 1), lambda i: (i, 0, 0)),
            pl.BlockSpec((block_b, N, D), lambda i: (i, 0, 0)),
            pl.BlockSpec((1, 1, D), lambda i: (0, 0, 0)),
        ],
        out_specs=pl.BlockSpec((block_b, N, D), lambda i: (i, 0, 0)),
        out_shape=jax.ShapeDtypeStruct((B, N, D), embedded_text.dtype),
    )(tok3, embedded_text, ph3)


def kernel(tokenized_text, embedded_text, placeholder_embedding):
    return _run(tokenized_text, embedded_text, placeholder_embedding)


# speedup vs baseline: 1.6660x; 1.0272x over previous
import functools

import jax
import jax.numpy as jnp
from jax.experimental import pallas as pl
from jax.experimental.pallas import tpu as pltpu

_PLACEHOLDER_TOKEN = 500
_NBUF = 8
_BLOCK_B = 8


def _stream_kernel(mask_hbm, emb_hbm, ph_hbm, out_hbm,
                   mask_vmem, ph_vmem, in_bufs, out_bufs,
                   mask_sem, ph_sem, in_sems, out_sems):
    nchunks = emb_hbm.shape[0] // _BLOCK_B

    def in_copy(chunk, slot):
        return pltpu.make_async_copy(
            emb_hbm.at[pl.ds(chunk * _BLOCK_B, _BLOCK_B)],
            in_bufs.at[slot],
            in_sems.at[slot],
        )

    def out_copy(chunk, slot):
        return pltpu.make_async_copy(
            out_bufs.at[slot],
            out_hbm.at[pl.ds(chunk * _BLOCK_B, _BLOCK_B)],
            out_sems.at[slot],
        )

    mask_cp = pltpu.make_async_copy(mask_hbm, mask_vmem, mask_sem)
    ph_cp = pltpu.make_async_copy(ph_hbm, ph_vmem, ph_sem)
    mask_cp.start()
    ph_cp.start()
    for k in range(_NBUF):
        in_copy(k, k).start()
    mask_cp.wait()
    ph_cp.wait()

    def body(i, _):
        slot = jax.lax.rem(i, _NBUF)
        in_copy(i, slot).wait()

        @pl.when(i >= _NBUF)
        def _wait_out():
            out_copy(i - _NBUF, slot).wait()

        m = mask_vmem[pl.ds(i * _BLOCK_B, _BLOCK_B)]
        out_bufs[slot] = jnp.where(m == 1, ph_vmem[...], in_bufs[slot])
        out_copy(i, slot).start()

        @pl.when(i + _NBUF < nchunks)
        def _refill():
            in_copy(i + _NBUF, slot).start()

        return 0

    jax.lax.fori_loop(0, nchunks, body, 0)
    for k in range(_NBUF):
        out_copy(nchunks - _NBUF + k, (nchunks - _NBUF + k) % _NBUF).wait()


@jax.jit
def _run(tokenized_text, embedded_text, placeholder_embedding):
    B, N, D = embedded_text.shape
    mask_u8 = (tokenized_text == _PLACEHOLDER_TOKEN).astype(jnp.int8)
    mask3 = mask_u8.reshape(B, N, 1)
    ph3 = placeholder_embedding.reshape(1, 1, D)
    return pl.pallas_call(
        _stream_kernel,
        in_specs=[
            pl.BlockSpec(memory_space=pltpu.MemorySpace.HBM),
            pl.BlockSpec(memory_space=pltpu.MemorySpace.HBM),
            pl.BlockSpec(memory_space=pltpu.MemorySpace.HBM),
        ],
        out_specs=pl.BlockSpec(memory_space=pltpu.MemorySpace.HBM),
        out_shape=jax.ShapeDtypeStruct((B, N, D), embedded_text.dtype),
        scratch_shapes=[
            pltpu.VMEM((B, N, 1), jnp.int8),
            pltpu.VMEM((1, 1, D), embedded_text.dtype),
            pltpu.VMEM((_NBUF, _BLOCK_B, N, D), embedded_text.dtype),
            pltpu.VMEM((_NBUF, _BLOCK_B, N, D), embedded_text.dtype),
            pltpu.SemaphoreType.DMA,
            pltpu.SemaphoreType.DMA,
            pltpu.SemaphoreType.DMA((_NBUF,)),
            pltpu.SemaphoreType.DMA((_NBUF,)),
        ],
    )(mask3, embedded_text, ph3)


def kernel(tokenized_text, embedded_text, placeholder_embedding):
    return _run(tokenized_text, embedded_text, placeholder_embedding)


# speedup vs baseline: 1.7223x; 1.0338x over previous
import jax
import jax.numpy as jnp
from jax.experimental import pallas as pl
from jax.experimental.pallas import tpu as pltpu

_PLACEHOLDER_TOKEN = 500
_NBUF = 8
_BLOCK_B = 8


def _stream_kernel(tok_hbm, emb_hbm, ph_hbm, out_hbm,
                   tok_vmem, ph_vmem, in_bufs, out_bufs,
                   tok_sem, ph_sem, in_sems, out_sems):
    nchunks = emb_hbm.shape[0] // _BLOCK_B

    def in_copy(chunk, slot):
        return pltpu.make_async_copy(
            emb_hbm.at[pl.ds(chunk * _BLOCK_B, _BLOCK_B)],
            in_bufs.at[slot],
            in_sems.at[slot],
        )

    def out_copy(chunk, slot):
        return pltpu.make_async_copy(
            out_bufs.at[slot],
            out_hbm.at[pl.ds(chunk * _BLOCK_B, _BLOCK_B)],
            out_sems.at[slot],
        )

    tok_cp = pltpu.make_async_copy(tok_hbm, tok_vmem, tok_sem)
    ph_cp = pltpu.make_async_copy(ph_hbm, ph_vmem, ph_sem)
    tok_cp.start()
    ph_cp.start()
    for k in range(_NBUF):
        in_copy(k, k).start()
    tok_cp.wait()
    ph_cp.wait()

    def body(i, _):
        slot = jax.lax.rem(i, _NBUF)
        in_copy(i, slot).wait()

        @pl.when(i >= _NBUF)
        def _wait_out():
            out_copy(i - _NBUF, slot).wait()

        tok3 = tok_vmem[pl.ds(i * _BLOCK_B, _BLOCK_B)][..., None]
        out_bufs[slot] = jnp.where(tok3 == _PLACEHOLDER_TOKEN,
                                   ph_vmem[...], in_bufs[slot])
        out_copy(i, slot).start()

        @pl.when(i + _NBUF < nchunks)
        def _refill():
            in_copy(i + _NBUF, slot).start()

        return 0

    jax.lax.fori_loop(0, nchunks, body, 0)
    for k in range(_NBUF):
        out_copy(nchunks - _NBUF + k, (nchunks - _NBUF + k) % _NBUF).wait()


@jax.jit
def _run(tokenized_text, embedded_text, placeholder_embedding):
    B, N, D = embedded_text.shape
    ph3 = placeholder_embedding.reshape(1, 1, D)
    return pl.pallas_call(
        _stream_kernel,
        in_specs=[
            pl.BlockSpec(memory_space=pltpu.MemorySpace.HBM),
            pl.BlockSpec(memory_space=pltpu.MemorySpace.HBM),
            pl.BlockSpec(memory_space=pltpu.MemorySpace.HBM),
        ],
        out_specs=pl.BlockSpec(memory_space=pltpu.MemorySpace.HBM),
        out_shape=jax.ShapeDtypeStruct((B, N, D), embedded_text.dtype),
        scratch_shapes=[
            pltpu.VMEM((B, N), jnp.int32),
            pltpu.VMEM((1, 1, D), embedded_text.dtype),
            pltpu.VMEM((_NBUF, _BLOCK_B, N, D), embedded_text.dtype),
            pltpu.VMEM((_NBUF, _BLOCK_B, N, D), embedded_text.dtype),
            pltpu.SemaphoreType.DMA,
            pltpu.SemaphoreType.DMA,
            pltpu.SemaphoreType.DMA((_NBUF,)),
            pltpu.SemaphoreType.DMA((_NBUF,)),
        ],
    )(tokenized_text, embedded_text, ph3)


def kernel(tokenized_text, embedded_text, placeholder_embedding):
    return _run(tokenized_text, embedded_text, placeholder_embedding)


# speedup vs baseline: 5.9717x; 3.4673x over previous
import jax
import jax.numpy as jnp
from jax.experimental import pallas as pl
from jax.experimental.pallas import tpu as pltpu

_PLACEHOLDER_TOKEN = 500
_NBUF = 6


def _stream_kernel(tok_hbm, emb_hbm, ph_hbm, out_hbm,
                   tok_vmem, ph_vmem, in_bufs, out_bufs,
                   tok_sem, ph_sem, in_sems, out_sems):
    nchunks = emb_hbm.shape[0]

    def in_copy(chunk, slot):
        return pltpu.make_async_copy(
            emb_hbm.at[pl.ds(chunk, 1)], in_bufs.at[slot], in_sems.at[slot])

    def out_copy(chunk, slot):
        return pltpu.make_async_copy(
            out_bufs.at[slot], out_hbm.at[pl.ds(chunk, 1)], out_sems.at[slot])

    tok_cp = pltpu.make_async_copy(tok_hbm, tok_vmem, tok_sem)
    ph_cp = pltpu.make_async_copy(ph_hbm, ph_vmem, ph_sem)
    tok_cp.start()
    ph_cp.start()
    for k in range(_NBUF):
        in_copy(k, k).start()
    tok_cp.wait()
    ph_cp.wait()

    def body(i, _):
        slot = jax.lax.rem(i, _NBUF)
        in_copy(i, slot).wait()

        @pl.when(i >= _NBUF)
        def _wait_out():
            out_copy(i - _NBUF, slot).wait()

        tok3 = tok_vmem[pl.ds(i, 1)][..., None]
        out_bufs[slot] = jnp.where(tok3 == _PLACEHOLDER_TOKEN,
                                   ph_vmem[...], in_bufs[slot])
        out_copy(i, slot).start()

        @pl.when(i + _NBUF < nchunks)
        def _refill():
            in_copy(i + _NBUF, slot).start()

        return 0

    jax.lax.fori_loop(0, nchunks, body, 0)
    for k in range(_NBUF):
        out_copy(nchunks - _NBUF + k, (nchunks - _NBUF + k) % _NBUF).wait()


@jax.jit
def _run(tokenized_text, embedded_text, placeholder_embedding):
    B, N, D = embedded_text.shape
    emb_t = embedded_text.transpose(1, 0, 2)
    tok_t = tokenized_text.transpose(1, 0)
    ph3 = placeholder_embedding.reshape(1, 1, D)
    out_t = pl.pallas_call(
        _stream_kernel,
        in_specs=[
            pl.BlockSpec(memory_space=pltpu.MemorySpace.HBM),
            pl.BlockSpec(memory_space=pltpu.MemorySpace.HBM),
            pl.BlockSpec(memory_space=pltpu.MemorySpace.HBM),
        ],
        out_specs=pl.BlockSpec(memory_space=pltpu.MemorySpace.HBM),
        out_shape=jax.ShapeDtypeStruct((N, B, D), embedded_text.dtype),
        scratch_shapes=[
            pltpu.VMEM((N, B), jnp.int32),
            pltpu.VMEM((1, 1, D), embedded_text.dtype),
            pltpu.VMEM((_NBUF, 1, B, D), embedded_text.dtype),
            pltpu.VMEM((_NBUF, 1, B, D), embedded_text.dtype),
            pltpu.SemaphoreType.DMA,
            pltpu.SemaphoreType.DMA,
            pltpu.SemaphoreType.DMA((_NBUF,)),
            pltpu.SemaphoreType.DMA((_NBUF,)),
        ],
    )(tok_t, emb_t, ph3)
    return out_t.transpose(1, 0, 2)


def kernel(tokenized_text, embedded_text, placeholder_embedding):
    return _run(tokenized_text, embedded_text, placeholder_embedding)


# speedup vs baseline: 5.9829x; 1.0019x over previous
import jax
import jax.numpy as jnp
from jax.experimental import pallas as pl
from jax.experimental.pallas import tpu as pltpu

_PLACEHOLDER_TOKEN = 500
_NBUF = 8


def _stream_kernel(tok_hbm, emb_hbm, ph_hbm, out_hbm,
                   tok_vmem, ph_vmem, in_bufs, out_bufs,
                   tok_sem, ph_sem, in_sems, out_sems):
    nchunks = emb_hbm.shape[0]

    def in_copy(chunk, slot):
        return pltpu.make_async_copy(
            emb_hbm.at[pl.ds(chunk, 1)], in_bufs.at[slot], in_sems.at[slot])

    def out_copy(chunk, slot):
        return pltpu.make_async_copy(
            out_bufs.at[slot], out_hbm.at[pl.ds(chunk, 1)], out_sems.at[slot])

    tok_cp = pltpu.make_async_copy(tok_hbm, tok_vmem, tok_sem)
    ph_cp = pltpu.make_async_copy(ph_hbm, ph_vmem, ph_sem)
    tok_cp.start()
    ph_cp.start()
    for k in range(_NBUF):
        in_copy(k, k).start()
    tok_cp.wait()
    ph_cp.wait()

    def body(i, _):
        slot = jax.lax.rem(i, _NBUF)
        in_copy(i, slot).wait()

        @pl.when(i >= _NBUF)
        def _wait_out():
            out_copy(i - _NBUF, slot).wait()

        tok3 = tok_vmem[pl.ds(i, 1)][..., None]
        out_bufs[slot] = jnp.where(tok3 == _PLACEHOLDER_TOKEN,
                                   ph_vmem[...], in_bufs[slot])
        out_copy(i, slot).start()

        @pl.when(i + _NBUF < nchunks)
        def _refill():
            in_copy(i + _NBUF, slot).start()

        return 0

    jax.lax.fori_loop(0, nchunks, body, 0)
    for k in range(_NBUF):
        out_copy(nchunks - _NBUF + k, (nchunks - _NBUF + k) % _NBUF).wait()


@jax.jit
def _run(tokenized_text, embedded_text, placeholder_embedding):
    B, N, D = embedded_text.shape
    emb_t = embedded_text.transpose(1, 0, 2)
    tok_t = tokenized_text.transpose(1, 0)
    ph3 = placeholder_embedding.reshape(1, 1, D)
    out_t = pl.pallas_call(
        _stream_kernel,
        in_specs=[
            pl.BlockSpec(memory_space=pltpu.MemorySpace.HBM),
            pl.BlockSpec(memory_space=pltpu.MemorySpace.HBM),
            pl.BlockSpec(memory_space=pltpu.MemorySpace.HBM),
        ],
        out_specs=pl.BlockSpec(memory_space=pltpu.MemorySpace.HBM),
        out_shape=jax.ShapeDtypeStruct((N, B, D), embedded_text.dtype),
        scratch_shapes=[
            pltpu.VMEM((N, B), jnp.int32),
            pltpu.VMEM((1, 1, D), embedded_text.dtype),
            pltpu.VMEM((_NBUF, 1, B, D), embedded_text.dtype),
            pltpu.VMEM((_NBUF, 1, B, D), embedded_text.dtype),
            pltpu.SemaphoreType.DMA,
            pltpu.SemaphoreType.DMA,
            pltpu.SemaphoreType.DMA((_NBUF,)),
            pltpu.SemaphoreType.DMA((_NBUF,)),
        ],
    )(tok_t, emb_t, ph3)
    return out_t.transpose(1, 0, 2)


def kernel(tokenized_text, embedded_text, placeholder_embedding):
    return _run(tokenized_text, embedded_text, placeholder_embedding)
